# baseline (speedup 1.0000x reference)
import functools

import jax
import jax.numpy as jnp
from jax import lax
from jax.experimental import pallas as pl
from jax.experimental.pallas import tpu as pltpu
from jax.experimental.pallas import tpu_sc as plsc

BATCH = 16384
NUM_WORKERS = 32
BPW = BATCH // NUM_WORKERS
CHUNK = 128
NCHUNK = BPW // CHUNK
D_MEM, D_PARTY, D_STATE = 32, 16, 16
D_OUT = D_MEM + D_PARTY + D_STATE
MEMBER_ROWS, STATE_ROWS, PARTY_ROWS = 1000000, 100000, 1000


def _sc_body(member_hbm, state_hbm, party_hbm,
             mtab_hbm, stab_hbm, ptab_hbm, out_hbm,
             midx_v, sidx_v, pidx_v, outbuf, ptab_v,
             msem, ssem, psem):
    wid = lax.axis_index("s") * 2 + lax.axis_index("c")
    base = wid * BPW
    row0 = wid * NCHUNK

    pltpu.sync_copy(member_hbm.at[pl.ds(row0, NCHUNK)], midx_v)
    pltpu.sync_copy(state_hbm.at[pl.ds(row0, NCHUNK)], sidx_v)
    pltpu.sync_copy(party_hbm.at[pl.ds(row0, NCHUNK)], pidx_v)
    pltpu.sync_copy(ptab_hbm, ptab_v)

    copies = []
    for j in range(NCHUNK):
        cols = pl.ds(j * CHUNK, CHUNK)
        for f in range(D_MEM):
            copies.append(pltpu.async_copy(
                mtab_hbm.at[f].at[midx_v.at[j]], outbuf.at[f, cols], msem))
        for f in range(D_STATE):
            copies.append(pltpu.async_copy(
                stab_hbm.at[f].at[sidx_v.at[j]],
                outbuf.at[D_MEM + D_PARTY + f, cols], ssem))

    def party_grp(it, _):
        f = it // (BPW // 16)
        g = it % (BPW // 16)
        pv = pidx_v[g // 8, pl.ds((g % 8) * 16, 16)]
        vals = plsc.load_gather(ptab_v, [jnp.full((16,), f, jnp.int32), pv])
        outbuf[D_MEM + f, pl.ds(g * 16, 16)] = vals
        return _

    lax.fori_loop(0, D_PARTY * (BPW // 16), party_grp, 0)

    for c in copies:
        c.wait()

    pltpu.sync_copy(outbuf, out_hbm.at[:, pl.ds(base, BPW)])


@jax.jit
def _mem_encoder_sc(member, state, party, member_table, state_table, party_table):
    mesh = plsc.VectorSubcoreMesh(core_axis_name="c", subcore_axis_name="s")
    k = functools.partial(
        pl.kernel,
        out_type=jax.ShapeDtypeStruct((D_OUT, BATCH), jnp.float32),
        mesh=mesh,
        scratch_types=[
            pltpu.VMEM((NCHUNK, CHUNK), jnp.int32),
            pltpu.VMEM((NCHUNK, CHUNK), jnp.int32),
            pltpu.VMEM((NCHUNK, CHUNK), jnp.int32),
            pltpu.VMEM((D_OUT, BPW), jnp.float32),
            pltpu.VMEM((D_PARTY, PARTY_ROWS), jnp.float32),
            pltpu.SemaphoreType.DMA,
            pltpu.SemaphoreType.DMA,
            pltpu.SemaphoreType.DMA,
        ],
        compiler_params=pltpu.CompilerParams(
            use_tc_tiling_on_sc=False, needs_layout_passes=False),
    )(_sc_body)
    member2d = member.astype(jnp.int32).reshape(BATCH // CHUNK, CHUNK)
    state2d = state.astype(jnp.int32).reshape(BATCH // CHUNK, CHUNK)
    party2d = party.astype(jnp.int32).reshape(BATCH // CHUNK, CHUNK)
    out_t = k(member2d, state2d, party2d,
              member_table.T, state_table.T, party_table.T)
    return out_t.T


def kernel(member, state, party, member_table, state_table, party_table):
    return _mem_encoder_sc(member, state, party,
                           member_table, state_table, party_table)
